# Initial kernel scaffold; baseline (speedup 1.0000x reference)
import jax
import jax.numpy as jnp
from jax.experimental import pallas as pl


def kernel(x, contig_ei, alliance_ei, trade_ei, W_contig, b_contig, W_alliance, b_alliance, W_trade, b_trade, ln1_w, ln1_b, W_conv2, b_conv2, ln2_w, ln2_b, Wl1, bl1, Wl2, bl2, Wf1, bf1, Wf2, bf2, Wh1, bh1, Wh2, bh2):
    raise NotImplementedError("write your pallas kernel here")


# speedup vs baseline: 20.6192x; 20.6192x over previous
import functools

import jax
import jax.numpy as jnp
from jax import lax
from jax.experimental import pallas as pl
from jax.experimental.pallas import tpu as pltpu
from jax.experimental.pallas import tpu_sc as plsc

_N = 10000
_D = 128
_H = 32
_T = 4
_O = 5
_E = 320000

_NC = 2
_NS = 16
_NW = _NC * _NS
_NPAD = 10240
_RPS = _NPAD // _NS
_CHUNK = 128
_EPAD = 327680
_NCH = _EPAD // (_NW * _CHUNK)


def _make_sc_scatter(nrel, width):
    mesh = plsc.VectorSubcoreMesh(core_axis_name="c", subcore_axis_name="s")
    scratch = [pltpu.VMEM_SHARED((_NPAD, width), jnp.float32)
               for _ in range(nrel)]
    scratch += [
        pltpu.VMEM((_NCH, _CHUNK), jnp.int32),
        pltpu.VMEM((_NCH, _CHUNK), jnp.int32),
        pltpu.VMEM((_CHUNK, width), jnp.float32),
        pltpu.SemaphoreType.DMA,
    ]

    @functools.partial(
        pl.kernel,
        mesh=mesh,
        out_type=jax.ShapeDtypeStruct((_NC, nrel, _NPAD, width), jnp.float32),
        scratch_types=scratch,
        compiler_params=pltpu.CompilerParams(use_tc_tiling_on_sc=False),
    )
    def sc_kernel(*refs):
        tables = refs[:nrel]
        src_hbm, dst_hbm, zeros_hbm, out_hbm = refs[nrel:nrel + 4]
        accs = refs[nrel + 4:2 * nrel + 4]
        src_v, dst_v, rows_v, sem = refs[2 * nrel + 4:]

        c = lax.axis_index("c")
        s = lax.axis_index("s")
        wid = c * _NS + s

        for acc in accs:
            pltpu.sync_copy(zeros_hbm.at[pl.ds(s * _RPS, _RPS)],
                            acc.at[pl.ds(s * _RPS, _RPS)])
        plsc.subcore_barrier()

        for r in range(nrel):
            pltpu.sync_copy(src_hbm.at[r, wid], src_v)
            pltpu.sync_copy(dst_hbm.at[r, wid], dst_v)
            table = tables[r]
            acc = accs[r]

            def body(j, carry, table=table, acc=acc):
                pltpu.async_copy(table.at[src_v.at[j]], rows_v, sem).wait()
                pltpu.sync_copy(rows_v, acc.at[dst_v.at[j]], add=True)
                return carry

            lax.fori_loop(0, _NCH, body, 0)
        plsc.subcore_barrier()

        for r in range(nrel):
            pltpu.sync_copy(accs[r].at[pl.ds(s * _RPS, _RPS)],
                            out_hbm.at[c, r, pl.ds(s * _RPS, _RPS)])

    return sc_kernel


_sc_scatter3_16 = _make_sc_scatter(3, 16)
_sc_scatter3_32 = _make_sc_scatter(3, _H)
_sc_scatter1_32 = _make_sc_scatter(1, _H)


def _elu(v):
    return jnp.where(v > 0, v, jnp.exp(v) - 1.0)


def _softmax(v):
    m = jnp.max(v, axis=-1, keepdims=True)
    e = jnp.exp(v - m)
    return e / jnp.sum(e, axis=-1, keepdims=True)


def _layernorm(h, w, b):
    mu = jnp.mean(h, axis=-1, keepdims=True)
    var = jnp.mean((h - mu) ** 2, axis=-1, keepdims=True)
    return (h - mu) * lax.rsqrt(var + 1e-5) * w + b


def _dot(a, b):
    return jnp.dot(a, b, preferred_element_type=jnp.float32)


def _row(ref):
    return ref[...].reshape(1, -1)


_BR = 1280


def _bs(shape):
    nd = len(shape)
    return pl.BlockSpec((_BR,) + tuple(shape[1:]),
                        lambda i: (i,) + (0,) * (nd - 1))


def _full(shape):
    nd = len(shape)
    return pl.BlockSpec(tuple(shape), lambda i: (0,) * nd)


def _tc1_body(x_ref, a0c_ref, a1c_ref, a0a_ref, a1a_ref, a0t_ref, a1t_ref,
              Wc_ref, bc_ref, Wa_ref, ba_ref, Wt_ref, bt_ref,
              Wl1_ref, bl1_ref, Wl2_ref, bl2_ref,
              Wh1_ref, bh1_ref, Wh2_ref, bh2_ref,
              Pc_ref, Pa_ref, Pt_ref, self_ref, dis_ref, expo_ref, yl_ref):
    x = x_ref[...]
    st = jnp.zeros((_BR, _H), jnp.float32) + _row(bc_ref) + _row(ba_ref) + _row(bt_ref)
    expos = []
    disl = []
    rels = [(a0c_ref, a1c_ref, Wc_ref, Pc_ref),
            (a0a_ref, a1a_ref, Wa_ref, Pa_ref),
            (a0t_ref, a1t_ref, Wt_ref, Pt_ref)]
    for a0_ref, a1_ref, W_ref, P_ref in rels:
        acc = a0_ref[...] + a1_ref[...]
        deg = acc[:, _T:_T + 1]
        dis = lax.rsqrt(deg + 1.0)
        xw = _dot(x, W_ref[...])
        P_ref[...] = xw * dis
        st = st + xw * (dis * dis)
        expos.append(acc[:, :_T] / jnp.maximum(deg, 1.0))
        disl.append(dis)
    self_ref[...] = st
    dis_ref[...] = jnp.concatenate(disl, axis=-1)
    expo_ref[...] = jnp.concatenate(expos, axis=-1)

    t = x[:, :_T]
    hl = _elu(_dot(x, Wl1_ref[...]) + _row(bl1_ref))
    hl = _elu(_dot(hl, Wl2_ref[...]) + _row(bl2_ref))
    zl = _elu(_dot(hl, Wh1_ref[:_H]) + _dot(t, Wh1_ref[_H:]) + _row(bh1_ref))
    yl_ref[...] = _softmax(_dot(zl, Wh2_ref[...]) + _row(bh2_ref))


def _tc1(xpad, parts, Wc, bc, Wa, ba, Wt, bt, Wl1, bl1, Wl2, bl2,
         Wh1, bh1, Wh2, bh2):
    args = (xpad, parts[0, 0], parts[1, 0], parts[0, 1], parts[1, 1],
            parts[0, 2], parts[1, 2],
            Wc, bc, Wa, ba, Wt, bt, Wl1, bl1, Wl2, bl2, Wh1, bh1, Wh2, bh2)
    out_shapes = [
        jax.ShapeDtypeStruct((_NPAD, _H), jnp.float32),
        jax.ShapeDtypeStruct((_NPAD, _H), jnp.float32),
        jax.ShapeDtypeStruct((_NPAD, _H), jnp.float32),
        jax.ShapeDtypeStruct((_NPAD, _H), jnp.float32),
        jax.ShapeDtypeStruct((_NPAD, 3), jnp.float32),
        jax.ShapeDtypeStruct((_NPAD, 3 * _T), jnp.float32),
        jax.ShapeDtypeStruct((_NPAD, _O), jnp.float32),
    ]
    in_specs = [_bs(a.shape) if a.shape[0] == _NPAD else _full(a.shape)
                for a in args]
    out_specs = [_bs(o.shape) for o in out_shapes]
    return pl.pallas_call(
        _tc1_body,
        grid=(_NPAD // _BR,),
        in_specs=in_specs,
        out_specs=out_specs,
        out_shape=out_shapes,
    )(*args)


def _tc2_body(a0c_ref, a1c_ref, a0a_ref, a1a_ref, a0t_ref, a1t_ref,
              self_ref, dis_ref, Wc2_ref, ln1w_ref, ln1b_ref,
              P2_ref, hw_ref):
    hpre = self_ref[...]
    rels = [(a0c_ref, a1c_ref, 0), (a0a_ref, a1a_ref, 1), (a0t_ref, a1t_ref, 2)]
    for a0_ref, a1_ref, r in rels:
        agg = a0_ref[...] + a1_ref[...]
        hpre = hpre + agg * dis_ref[:, r:r + 1]
    h = _elu(_layernorm(hpre, _row(ln1w_ref), _row(ln1b_ref)))
    hw = _dot(h, Wc2_ref[...])
    hw_ref[...] = hw
    P2_ref[...] = hw * dis_ref[:, 0:1]


def _tc2(parts, selfterm, dis, Wc2, ln1w, ln1b):
    args = (parts[0, 0], parts[1, 0], parts[0, 1], parts[1, 1], parts[0, 2],
            parts[1, 2], selfterm, dis, Wc2, ln1w, ln1b)
    out_shapes = [
        jax.ShapeDtypeStruct((_NPAD, _H), jnp.float32),
        jax.ShapeDtypeStruct((_NPAD, _H), jnp.float32),
    ]
    in_specs = [_bs(a.shape) if a.shape[0] == _NPAD else _full(a.shape)
                for a in args]
    return pl.pallas_call(
        _tc2_body,
        grid=(_NPAD // _BR,),
        in_specs=in_specs,
        out_specs=[_bs(o.shape) for o in out_shapes],
        out_shape=out_shapes,
    )(*args)


def _tc3_body(a0_ref, a1_ref, hw_ref, dis_ref, bc2_ref, ln2w_ref, ln2b_ref,
              x_ref, expo_ref, Wf1_ref, bf1_ref, Wf2_ref, bf2_ref, yf_ref):
    agg2 = a0_ref[...] + a1_ref[...]
    dis = dis_ref[:, 0:1]
    g = agg2 * dis + hw_ref[...] * dis * dis + _row(bc2_ref)
    h2 = _elu(_layernorm(g, _row(ln2w_ref), _row(ln2b_ref)))
    t = x_ref[:, :_T]
    expo = expo_ref[...]
    z = _elu(_dot(h2, Wf1_ref[:_H]) + _dot(t, Wf1_ref[_H:_H + _T])
             + _dot(expo, Wf1_ref[_H + _T:]) + _row(bf1_ref))
    yf_ref[...] = _softmax(_dot(z, Wf2_ref[...]) + _row(bf2_ref))


def _tc3(parts2, hw, dis, bc2, ln2w, ln2b, xpad, expo, Wf1, bf1, Wf2, bf2):
    args = (parts2[0, 0], parts2[1, 0], hw, dis, bc2, ln2w, ln2b, xpad, expo,
            Wf1, bf1, Wf2, bf2)
    out_shape = jax.ShapeDtypeStruct((_NPAD, _O), jnp.float32)
    in_specs = [_bs(a.shape) if a.shape[0] == _NPAD else _full(a.shape)
                for a in args]
    return pl.pallas_call(
        _tc3_body,
        grid=(_NPAD // _BR,),
        in_specs=in_specs,
        out_specs=_bs(out_shape.shape),
        out_shape=out_shape,
    )(*args)


def kernel(x, contig_ei, alliance_ei, trade_ei, W_contig, b_contig,
           W_alliance, b_alliance, W_trade, b_trade, ln1_w, ln1_b,
           W_conv2, b_conv2, ln2_w, ln2_b, Wl1, bl1, Wl2, bl2,
           Wf1, bf1, Wf2, bf2, Wh1, bh1, Wh2, bh2):
    f32 = jnp.float32
    xpad = jnp.pad(x.astype(f32), ((0, _NPAD - _N), (0, 0)))
    t = xpad[:, :_T]
    ones = jnp.concatenate(
        [jnp.ones((_N, 1), f32), jnp.zeros((_NPAD - _N, 1), f32)], axis=0)
    texp = jnp.concatenate([t, ones, jnp.zeros((_NPAD, 11), f32)], axis=-1)

    def prep(ei):
        pad = jnp.full((2, _EPAD - _E), _N, jnp.int32)
        e = jnp.concatenate([ei.astype(jnp.int32), pad], axis=1)
        return e.reshape(2, _NW, _NCH, _CHUNK)

    e_c, e_a, e_t = prep(contig_ei), prep(alliance_ei), prep(trade_ei)
    src3 = jnp.stack([e_c[0], e_a[0], e_t[0]])
    dst3 = jnp.stack([e_c[1], e_a[1], e_t[1]])
    zeros16 = jnp.zeros((_NPAD, 16), f32)
    zeros32 = jnp.zeros((_NPAD, _H), f32)

    acc1 = _sc_scatter3_16(texp, texp, texp, src3, dst3, zeros16)

    Pc, Pa, Pt, selfterm, dis, expo, y_local = _tc1(
        xpad, acc1, W_contig, b_contig, W_alliance, b_alliance, W_trade,
        b_trade, Wl1, bl1, Wl2, bl2, Wh1, bh1, Wh2, bh2)

    agg = _sc_scatter3_32(Pc, Pa, Pt, src3, dst3, zeros32)

    P2, hw = _tc2(agg, selfterm, dis, W_conv2, ln1_w, ln1_b)

    agg2 = _sc_scatter1_32(P2, src3[0:1], dst3[0:1], zeros32)

    y_full = _tc3(agg2, hw, dis, b_conv2, ln2_w, ln2_b, xpad, expo,
                  Wf1, bf1, Wf2, bf2)

    return (y_full[:_N], y_local[:_N], expo[:_N])
